# Initial kernel scaffold; baseline (speedup 1.0000x reference)
import jax
import jax.numpy as jnp
from jax.experimental import pallas as pl


def kernel(x, edge_index, W1_0, b1_0, g_0, bt_0, rm_0, rv_0, W2_0, b2_0, W1_1, b1_1, g_1, bt_1, rm_1, rv_1, W2_1, b2_1, W1_2, b1_2, g_2, bt_2, rm_2, rv_2, W2_2, b2_2, eps_0, eps_1):
    raise NotImplementedError("write your pallas kernel here")


# speedup vs baseline: 5.8000x; 5.8000x over previous
import functools

import jax
import jax.numpy as jnp
from jax import lax
from jax.experimental import pallas as pl
from jax.experimental.pallas import tpu as pltpu
from jax.experimental.pallas import tpu_sc as plsc

NSUB = 16
NCORE = 2
CH = 128
ZR = 64


@functools.lru_cache(maxsize=None)
def _make_aggr(N, NSB, SB, ACC_ROWS):
  mesh = plsc.VectorSubcoreMesh(core_axis_name="c", subcore_axis_name="s")
  RPZ = ACC_ROWS // NSUB
  FR = (N // NSUB) // 8 * 8
  TAIL = N - NSUB * FR

  @functools.partial(
      pl.kernel,
      out_type=jax.ShapeDtypeStruct((2 * N, 128), jnp.float32),
      mesh=mesh,
      scratch_types=[
          pltpu.VMEM((SB, CH), jnp.int32),
          pltpu.VMEM((SB, CH), jnp.int32),
          pltpu.VMEM((CH, 128), jnp.float32),
          pltpu.VMEM((CH, 128), jnp.float32),
          pltpu.VMEM_SHARED((ACC_ROWS, 128), jnp.float32),
          pltpu.SemaphoreType.DMA,
          pltpu.SemaphoreType.DMA,
      ],
  )
  def aggr(h2, src3, dst3, out, isrc, idst, rows0, rows1, acc, sem0, sem1):
    cid = lax.axis_index("c")
    sid = lax.axis_index("s")

    zv = jnp.zeros((16,), jnp.float32)

    def zfill(i, _):
      for k in range(8):
        rows0[i, pl.ds(k * 16, 16)] = zv
      return 0

    lax.fori_loop(0, CH, zfill, 0)

    zbase = sid * RPZ
    nfull = RPZ // CH

    def zcopy(i, _):
      pltpu.sync_copy(rows0, acc.at[pl.ds(zbase + i * CH, CH)])
      return 0

    lax.fori_loop(0, nfull, zcopy, 0)
    rem = RPZ - nfull * CH
    if rem:
      pltpu.sync_copy(rows0.at[pl.ds(0, rem)],
                      acc.at[pl.ds(zbase + nfull * CH, rem)])
    plsc.subcore_barrier()

    rows = (rows0, rows1)
    sems = (sem0, sem1)

    def body(k, _):
      pltpu.sync_copy(src3.at[cid, sid, k], isrc)
      pltpu.sync_copy(dst3.at[sid, k], idst)
      pltpu.async_copy(h2.at[isrc.at[0]], rows0, sem0)
      for j in range(SB):
        if j + 1 < SB:
          pltpu.async_copy(h2.at[isrc.at[j + 1]], rows[(j + 1) % 2],
                           sems[(j + 1) % 2])
        pltpu.make_async_copy(h2.at[isrc.at[j]], rows[j % 2],
                              sems[j % 2]).wait()
        pltpu.sync_copy(rows[j % 2], acc.at[idst.at[j]], add=True)
      return 0

    lax.fori_loop(0, NSB, body, 0)

    plsc.subcore_barrier()
    pltpu.sync_copy(acc.at[pl.ds(sid * FR, FR)],
                    out.at[pl.ds(cid * N + sid * FR, FR)])
    if TAIL:
      @pl.when(sid == NSUB - 1)
      def _():
        pltpu.sync_copy(acc.at[pl.ds(NSUB * FR, TAIL)],
                        out.at[pl.ds(cid * N + NSUB * FR, TAIL)])

  return aggr


def _mlp01_body(s_ref, h_ref, a_ref, w1_ref, b1_ref, w2_ref, b2_ref, o_ref):
  h = jnp.concatenate([h_ref[0], h_ref[1]], axis=-1)
  a = jnp.concatenate([a_ref[0], a_ref[1]], axis=-1)
  z = h * s_ref[0, 0] + a
  t = jnp.dot(z, w1_ref[...], preferred_element_type=jnp.float32)
  t = jnp.maximum(t + b1_ref[...], 0.0)
  o = jnp.dot(t, w2_ref[...], preferred_element_type=jnp.float32)
  o = jnp.maximum(o + b2_ref[...], 0.0)
  o_ref[0] = o[:, :128]
  o_ref[1] = o[:, 128:]


def _mlp2_body(h_ref, a_ref, w1_ref, b1_ref, w2_ref, b2_ref, o_ref):
  h = jnp.concatenate([h_ref[0], h_ref[1]], axis=-1)
  a = jnp.concatenate([a_ref[0], a_ref[1]], axis=-1)
  z = 2.0 * h + a
  t = jnp.dot(z, w1_ref[...], preferred_element_type=jnp.float32)
  t = jnp.maximum(t + b1_ref[...], 0.0)
  o = jnp.dot(t, w2_ref[...], preferred_element_type=jnp.float32)
  o = o + b2_ref[...]
  m = jnp.max(o, axis=-1, keepdims=True)
  lse = jnp.log(jnp.sum(jnp.exp(o - m), axis=-1, keepdims=True)) + m
  o_ref[...] = o - lse


@functools.lru_cache(maxsize=None)
def _make_mlp01(N, H, B):
  grid = (N // B,)
  return pl.pallas_call(
      _mlp01_body,
      grid=grid,
      in_specs=[
          pl.BlockSpec(memory_space=pltpu.SMEM),
          pl.BlockSpec((2, B, 128), lambda i: (0, i, 0)),
          pl.BlockSpec((2, B, 128), lambda i: (0, i, 0)),
          pl.BlockSpec((H, H), lambda i: (0, 0)),
          pl.BlockSpec((1, H), lambda i: (0, 0)),
          pl.BlockSpec((H, H), lambda i: (0, 0)),
          pl.BlockSpec((1, H), lambda i: (0, 0)),
      ],
      out_specs=pl.BlockSpec((2, B, 128), lambda i: (0, i, 0)),
      out_shape=jax.ShapeDtypeStruct((2, N, 128), jnp.float32),
  )


@functools.lru_cache(maxsize=None)
def _make_mlp2(N, H, B):
  grid = (N // B,)
  return pl.pallas_call(
      _mlp2_body,
      grid=grid,
      in_specs=[
          pl.BlockSpec((2, B, 128), lambda i: (0, i, 0)),
          pl.BlockSpec((2, B, 128), lambda i: (0, i, 0)),
          pl.BlockSpec((H, 128), lambda i: (0, 0)),
          pl.BlockSpec((1, 128), lambda i: (0, 0)),
          pl.BlockSpec((128, 128), lambda i: (0, 0)),
          pl.BlockSpec((1, 128), lambda i: (0, 0)),
      ],
      out_specs=pl.BlockSpec((B, 128), lambda i: (i, 0)),
      out_shape=jax.ShapeDtypeStruct((N, 128), jnp.float32),
  )


def _fold_bn(W1, b1, g, bt, rm, rv):
  sc = g * lax.rsqrt(rv + 1e-5)
  return W1 * sc[None, :], ((b1 - rm) * sc + bt)[None, :]


def kernel(x, edge_index, W1_0, b1_0, g_0, bt_0, rm_0, rv_0, W2_0, b2_0,
           W1_1, b1_1, g_1, bt_1, rm_1, rv_1, W2_1, b2_1,
           W1_2, b1_2, g_2, bt_2, rm_2, rv_2, W2_2, b2_2, eps_0, eps_1):
  N, D = x.shape
  E = edge_index.shape[1]
  H = W1_0.shape[1]
  OUT = W1_2.shape[1]
  assert D == 256 and H == 256 and N % NSUB == 0

  src, dst = edge_index[0], edge_index[1]
  src_all = jnp.concatenate([src, dst])
  dst_all = jnp.concatenate([dst, src])
  E2 = 2 * E
  SB = 16
  NSB = -(-E2 // (NSUB * SB * CH))
  pad = NSUB * NSB * SB * CH - E2
  srcp = jnp.concatenate([src_all, jnp.zeros((pad,), jnp.int32)])
  dstp = jnp.concatenate([dst_all, jnp.full((pad,), N, jnp.int32)])
  src_r = srcp.reshape(NSUB, NSB, SB, CH)
  src3 = jnp.stack([src_r, src_r + N])
  dst3 = dstp.reshape(NSUB, NSB, SB, CH)

  ACC_ROWS = ((N + 1 + NSUB * 8 - 1) // (NSUB * 8)) * (NSUB * 8)
  aggr_fn = _make_aggr(N, NSB, SB, ACC_ROWS)

  W1f0, b1f0 = _fold_bn(W1_0, b1_0, g_0, bt_0, rm_0, rv_0)
  W1f1, b1f1 = _fold_bn(W1_1, b1_1, g_1, bt_1, rm_1, rv_1)
  W1f2, b1f2 = _fold_bn(W1_2, b1_2, g_2, bt_2, rm_2, rv_2)
  W1p = jnp.zeros((H, 128), jnp.float32).at[:, :OUT].set(W1f2)
  b1p = jnp.zeros((1, 128), jnp.float32).at[:, :OUT].set(b1f2)
  W2p = jnp.zeros((128, 128), jnp.float32).at[:OUT, :OUT].set(W2_2)
  b2p = jnp.full((1, 128), -1e30, jnp.float32).at[:, :OUT].set(b2_2[None, :])
  s0 = jnp.reshape(2.0 + eps_0, (1, 1))
  s1 = jnp.reshape(2.0 + eps_1, (1, 1))

  B = 1000
  mlp01 = _make_mlp01(N, H, B)
  mlp2 = _make_mlp2(N, H, B)

  h = jnp.stack([x[:, :128], x[:, 128:]])
  a = aggr_fn(h.reshape(2 * N, 128), src3, dst3)
  h = mlp01(s0, h, a.reshape(2, N, 128), W1f0, b1f0, W2_0, b2_0[None, :])
  a = aggr_fn(h.reshape(2 * N, 128), src3, dst3)
  h = mlp01(s1, h, a.reshape(2, N, 128), W1f1, b1f1, W2_1, b2_1[None, :])
  a = aggr_fn(h.reshape(2 * N, 128), src3, dst3)
  o = mlp2(h, a.reshape(2, N, 128), W1p, b1p, W2p, b2p)
  return o[:, :OUT]


# speedup vs baseline: 13.1693x; 2.2706x over previous
import functools

import jax
import jax.numpy as jnp
from jax import lax
from jax.experimental import pallas as pl
from jax.experimental.pallas import tpu as pltpu
from jax.experimental.pallas import tpu_sc as plsc

NSUB = 16
NCORE = 2
CH = 128
ZR = 64


@functools.lru_cache(maxsize=None)
def _make_aggr(N, NSB, SB, ACC_ROWS):
  mesh = plsc.VectorSubcoreMesh(core_axis_name="c", subcore_axis_name="s")
  RPZ = ACC_ROWS // NSUB
  FR = (N // NSUB) // 8 * 8
  TAIL = N - NSUB * FR

  @functools.partial(
      pl.kernel,
      out_type=jax.ShapeDtypeStruct((2 * N, 128), jnp.float32),
      mesh=mesh,
      scratch_types=[
          pltpu.VMEM((SB, CH), jnp.int32),
          pltpu.VMEM((SB, CH), jnp.int32),
          pltpu.VMEM((CH, 128), jnp.float32),
          pltpu.VMEM((CH, 128), jnp.float32),
          pltpu.VMEM_SHARED((ACC_ROWS, 128), jnp.float32),
          pltpu.SemaphoreType.DMA,
          pltpu.SemaphoreType.DMA,
      ],
  )
  def aggr(h2, src3, dst3, out, isrc, idst, rows0, rows1, acc, sem0, sem1):
    cid = lax.axis_index("c")
    sid = lax.axis_index("s")

    zv = jnp.zeros((16,), jnp.float32)

    def zfill(i, _):
      for k in range(8):
        rows0[i, pl.ds(k * 16, 16)] = zv
      return 0

    lax.fori_loop(0, CH, zfill, 0)

    zbase = sid * RPZ
    nfull = RPZ // CH

    def zcopy(i, _):
      pltpu.sync_copy(rows0, acc.at[pl.ds(zbase + i * CH, CH)])
      return 0

    lax.fori_loop(0, nfull, zcopy, 0)
    rem = RPZ - nfull * CH
    if rem:
      pltpu.sync_copy(rows0.at[pl.ds(0, rem)],
                      acc.at[pl.ds(zbase + nfull * CH, rem)])
    plsc.subcore_barrier()

    rows = (rows0, rows1)
    sems = (sem0, sem1)

    def body(k, _):
      pltpu.sync_copy(src3.at[cid, sid, k], isrc)
      pltpu.sync_copy(dst3.at[sid, k], idst)
      pltpu.async_copy(h2.at[isrc.at[0]], rows0, sem0)
      for j in range(SB):
        if j + 1 < SB:
          pltpu.async_copy(h2.at[isrc.at[j + 1]], rows[(j + 1) % 2],
                           sems[(j + 1) % 2])
        pltpu.make_async_copy(h2.at[isrc.at[j]], rows[j % 2],
                              sems[j % 2]).wait()
        pltpu.sync_copy(rows[j % 2], acc.at[idst.at[j]], add=True)
      return 0

    lax.fori_loop(0, NSB, body, 0)

    plsc.subcore_barrier()
    pltpu.sync_copy(acc.at[pl.ds(sid * FR, FR)],
                    out.at[pl.ds(cid * N + sid * FR, FR)])
    if TAIL:
      @pl.when(sid == NSUB - 1)
      def _():
        pltpu.sync_copy(acc.at[pl.ds(NSUB * FR, TAIL)],
                        out.at[pl.ds(cid * N + NSUB * FR, TAIL)])

  return aggr


def _mlp01_body(s_ref, h_ref, a_ref, w1_ref, b1_ref, w2_ref, b2_ref, o_ref):
  h = jnp.concatenate([h_ref[0], h_ref[1]], axis=-1)
  a = jnp.concatenate([a_ref[0], a_ref[1]], axis=-1)
  z = h * s_ref[0, 0] + a
  t = jnp.dot(z, w1_ref[...], preferred_element_type=jnp.float32)
  t = jnp.maximum(t + b1_ref[...], 0.0)
  o = jnp.dot(t, w2_ref[...], preferred_element_type=jnp.float32)
  o = jnp.maximum(o + b2_ref[...], 0.0)
  o_ref[0] = o[:, :128]
  o_ref[1] = o[:, 128:]


def _mlp2_body(h_ref, a_ref, w1_ref, b1_ref, w2_ref, b2_ref, o_ref):
  h = jnp.concatenate([h_ref[0], h_ref[1]], axis=-1)
  a = jnp.concatenate([a_ref[0], a_ref[1]], axis=-1)
  z = 2.0 * h + a
  t = jnp.dot(z, w1_ref[...], preferred_element_type=jnp.float32)
  t = jnp.maximum(t + b1_ref[...], 0.0)
  o = jnp.dot(t, w2_ref[...], preferred_element_type=jnp.float32)
  o = o + b2_ref[...]
  m = jnp.max(o, axis=-1, keepdims=True)
  lse = jnp.log(jnp.sum(jnp.exp(o - m), axis=-1, keepdims=True)) + m
  o_ref[...] = o - lse


@functools.lru_cache(maxsize=None)
def _make_mlp01(N, H, B):
  grid = (N // B,)
  return pl.pallas_call(
      _mlp01_body,
      grid=grid,
      in_specs=[
          pl.BlockSpec(memory_space=pltpu.SMEM),
          pl.BlockSpec((2, B, 128), lambda i: (0, i, 0)),
          pl.BlockSpec((2, B, 128), lambda i: (0, i, 0)),
          pl.BlockSpec((H, H), lambda i: (0, 0)),
          pl.BlockSpec((1, H), lambda i: (0, 0)),
          pl.BlockSpec((H, H), lambda i: (0, 0)),
          pl.BlockSpec((1, H), lambda i: (0, 0)),
      ],
      out_specs=pl.BlockSpec((2, B, 128), lambda i: (0, i, 0)),
      out_shape=jax.ShapeDtypeStruct((2, N, 128), jnp.float32),
  )


@functools.lru_cache(maxsize=None)
def _make_mlp2(N, H, B):
  grid = (N // B,)
  return pl.pallas_call(
      _mlp2_body,
      grid=grid,
      in_specs=[
          pl.BlockSpec((2, B, 128), lambda i: (0, i, 0)),
          pl.BlockSpec((2, B, 128), lambda i: (0, i, 0)),
          pl.BlockSpec((H, 128), lambda i: (0, 0)),
          pl.BlockSpec((1, 128), lambda i: (0, 0)),
          pl.BlockSpec((128, 128), lambda i: (0, 0)),
          pl.BlockSpec((1, 128), lambda i: (0, 0)),
      ],
      out_specs=pl.BlockSpec((B, 128), lambda i: (i, 0)),
      out_shape=jax.ShapeDtypeStruct((N, 128), jnp.float32),
  )


def _fold_bn(W1, b1, g, bt, rm, rv):
  sc = g * lax.rsqrt(rv + 1e-5)
  return W1 * sc[None, :], ((b1 - rm) * sc + bt)[None, :]


def kernel(x, edge_index, W1_0, b1_0, g_0, bt_0, rm_0, rv_0, W2_0, b2_0,
           W1_1, b1_1, g_1, bt_1, rm_1, rv_1, W2_1, b2_1,
           W1_2, b1_2, g_2, bt_2, rm_2, rv_2, W2_2, b2_2, eps_0, eps_1):
  N, D = x.shape
  E = edge_index.shape[1]
  H = W1_0.shape[1]
  OUT = W1_2.shape[1]
  assert D == 256 and H == 256 and N % NSUB == 0

  src, dst = edge_index[0], edge_index[1]
  src_all = jnp.concatenate([src, dst])
  dst_all = jnp.concatenate([dst, src])
  E2 = 2 * E
  SB = 16
  NSB = -(-E2 // (NSUB * SB * CH))
  pad = NSUB * NSB * SB * CH - E2
  ACC_ROWS = ((N + 1 + NSUB * 8 - 1) // (NSUB * 8)) * (NSUB * 8)
  ar = jnp.arange(pad, dtype=jnp.int32)
  srcp = jnp.concatenate([src_all, ar % N])
  dstp = jnp.concatenate([dst_all, N + ar % (ACC_ROWS - N)])
  src_r = srcp.reshape(NSUB, NSB, SB, CH)
  src3 = jnp.stack([src_r, src_r + N])
  dst3 = dstp.reshape(NSUB, NSB, SB, CH)
  aggr_fn = _make_aggr(N, NSB, SB, ACC_ROWS)

  W1f0, b1f0 = _fold_bn(W1_0, b1_0, g_0, bt_0, rm_0, rv_0)
  W1f1, b1f1 = _fold_bn(W1_1, b1_1, g_1, bt_1, rm_1, rv_1)
  W1f2, b1f2 = _fold_bn(W1_2, b1_2, g_2, bt_2, rm_2, rv_2)
  W1p = jnp.zeros((H, 128), jnp.float32).at[:, :OUT].set(W1f2)
  b1p = jnp.zeros((1, 128), jnp.float32).at[:, :OUT].set(b1f2)
  W2p = jnp.zeros((128, 128), jnp.float32).at[:OUT, :OUT].set(W2_2)
  b2p = jnp.full((1, 128), -1e30, jnp.float32).at[:, :OUT].set(b2_2[None, :])
  s0 = jnp.reshape(2.0 + eps_0, (1, 1))
  s1 = jnp.reshape(2.0 + eps_1, (1, 1))

  B = 1000
  mlp01 = _make_mlp01(N, H, B)
  mlp2 = _make_mlp2(N, H, B)

  h = jnp.stack([x[:, :128], x[:, 128:]])
  a = aggr_fn(h.reshape(2 * N, 128), src3, dst3)
  h = mlp01(s0, h, a.reshape(2, N, 128), W1f0, b1f0, W2_0, b2_0[None, :])
  a = aggr_fn(h.reshape(2 * N, 128), src3, dst3)
  h = mlp01(s1, h, a.reshape(2, N, 128), W1f1, b1f1, W2_1, b2_1[None, :])
  a = aggr_fn(h.reshape(2 * N, 128), src3, dst3)
  o = mlp2(h, a.reshape(2, N, 128), W1p, b1p, W2p, b2p)
  return o[:, :OUT]


# speedup vs baseline: 14.2903x; 1.0851x over previous
import functools

import jax
import jax.numpy as jnp
from jax import lax
from jax.experimental import pallas as pl
from jax.experimental.pallas import tpu as pltpu
from jax.experimental.pallas import tpu_sc as plsc

NSUB = 16
NCORE = 2
CH = 128
ZR = 64


@functools.lru_cache(maxsize=None)
def _make_aggr(N, NSB, SB, ACC_ROWS):
  mesh = plsc.VectorSubcoreMesh(core_axis_name="c", subcore_axis_name="s")
  FR = (N // NSUB) // 8 * 8
  TAIL = N - NSUB * FR
  NZ = 4
  ZR = ACC_ROWS // NZ
  assert NSB % 2 == 0 and ACC_ROWS % NZ == 0 and ZR % 8 == 0

  @functools.partial(
      pl.kernel,
      out_type=jax.ShapeDtypeStruct((2 * N, 128), jnp.float32),
      mesh=mesh,
      scratch_types=[
          pltpu.VMEM((SB, CH), jnp.int32),
          pltpu.VMEM((SB, CH), jnp.int32),
          pltpu.VMEM((SB, CH), jnp.int32),
          pltpu.VMEM((SB, CH), jnp.int32),
          pltpu.VMEM((CH, 128), jnp.float32),
          pltpu.VMEM((CH, 128), jnp.float32),
          pltpu.VMEM_SHARED((ACC_ROWS, 128), jnp.float32),
          pltpu.SemaphoreType.DMA,
          pltpu.SemaphoreType.DMA,
          pltpu.SemaphoreType.DMA,
          pltpu.SemaphoreType.DMA,
      ],
  )
  def aggr(h2, src3, dst3, zeros, out, isrcA, isrcB, idstA, idstB,
           rows0, rows1, acc, gsem0, gsem1, isem, zsem):
    cid = lax.axis_index("c")
    sid = lax.axis_index("s")
    rows = (rows0, rows1)
    gsem = (gsem0, gsem1)
    isrcs = (isrcA, isrcB)
    idsts = (idstA, idstB)

    @pl.when(sid < NZ)
    def _():
      pltpu.async_copy(zeros.at[pl.ds(sid * ZR, ZR)],
                       acc.at[pl.ds(sid * ZR, ZR)], zsem)
    pltpu.sync_copy(src3.at[cid, sid, 0], isrcA)
    pltpu.sync_copy(dst3.at[sid, 0], idstA)
    pltpu.async_copy(src3.at[cid, sid, 1], isrcB, isem)
    pltpu.async_copy(dst3.at[sid, 1], idstB, isem)
    pltpu.async_copy(h2.at[isrcA.at[0]], rows0, gsem0)
    pltpu.async_copy(h2.at[isrcA.at[1]], rows1, gsem1)
    @pl.when(sid < NZ)
    def _():
      pltpu.make_async_copy(zeros.at[pl.ds(0, ZR)],
                            acc.at[pl.ds(0, ZR)], zsem).wait()
    plsc.subcore_barrier()

    def pair(k2, _):
      for s in range(2):
        k = 2 * k2 + s
        cur_isrc, cur_idst = isrcs[s], idsts[s]
        nxt_isrc, nxt_idst = isrcs[1 - s], idsts[1 - s]
        for j in range(SB):
          b = j % 2
          if j == SB - 2:
            pltpu.make_async_copy(src3.at[cid, sid, 0], nxt_isrc, isem).wait()
            pltpu.make_async_copy(dst3.at[sid, 0], nxt_idst, isem).wait()
          pltpu.make_async_copy(h2.at[cur_isrc.at[j]], rows[b],
                                gsem[b]).wait()
          pltpu.sync_copy(rows[b], acc.at[cur_idst.at[j]], add=True)
          if j < SB - 2:
            pltpu.async_copy(h2.at[cur_isrc.at[j + 2]], rows[b], gsem[b])
          else:
            pltpu.async_copy(h2.at[nxt_isrc.at[j + 2 - SB]], rows[b], gsem[b])
        knx = jnp.minimum(k + 2, NSB - 1)
        pltpu.async_copy(src3.at[cid, sid, knx], cur_isrc, isem)
        pltpu.async_copy(dst3.at[sid, knx], cur_idst, isem)
      return 0

    lax.fori_loop(0, NSB // 2, pair, 0)

    pltpu.make_async_copy(h2.at[isrcA.at[0]], rows0, gsem0).wait()
    pltpu.make_async_copy(h2.at[isrcA.at[1]], rows1, gsem1).wait()
    pltpu.make_async_copy(src3.at[cid, sid, 0], isrcB, isem).wait()
    pltpu.make_async_copy(dst3.at[sid, 0], idstB, isem).wait()

    plsc.subcore_barrier()
    pltpu.sync_copy(acc.at[pl.ds(sid * FR, FR)],
                    out.at[pl.ds(cid * N + sid * FR, FR)])
    if TAIL:
      @pl.when(sid == NSUB - 1)
      def _():
        pltpu.sync_copy(acc.at[pl.ds(NSUB * FR, TAIL)],
                        out.at[pl.ds(cid * N + NSUB * FR, TAIL)])

  return aggr


def _mlp01_body(s_ref, h_ref, a_ref, w1_ref, b1_ref, w2_ref, b2_ref, o_ref):
  h = jnp.concatenate([h_ref[0], h_ref[1]], axis=-1)
  a = jnp.concatenate([a_ref[0], a_ref[1]], axis=-1)
  z = h * s_ref[0, 0] + a
  t = jnp.dot(z, w1_ref[...], preferred_element_type=jnp.float32)
  t = jnp.maximum(t + b1_ref[...], 0.0)
  o = jnp.dot(t, w2_ref[...], preferred_element_type=jnp.float32)
  o = jnp.maximum(o + b2_ref[...], 0.0)
  o_ref[0] = o[:, :128]
  o_ref[1] = o[:, 128:]


def _mlp2_body(h_ref, a_ref, w1_ref, b1_ref, w2_ref, b2_ref, o_ref):
  h = jnp.concatenate([h_ref[0], h_ref[1]], axis=-1)
  a = jnp.concatenate([a_ref[0], a_ref[1]], axis=-1)
  z = 2.0 * h + a
  t = jnp.dot(z, w1_ref[...], preferred_element_type=jnp.float32)
  t = jnp.maximum(t + b1_ref[...], 0.0)
  o = jnp.dot(t, w2_ref[...], preferred_element_type=jnp.float32)
  o = o + b2_ref[...]
  m = jnp.max(o, axis=-1, keepdims=True)
  lse = jnp.log(jnp.sum(jnp.exp(o - m), axis=-1, keepdims=True)) + m
  o_ref[...] = o - lse


@functools.lru_cache(maxsize=None)
def _make_mlp01(N, H, B):
  grid = (N // B,)
  return pl.pallas_call(
      _mlp01_body,
      grid=grid,
      in_specs=[
          pl.BlockSpec(memory_space=pltpu.SMEM),
          pl.BlockSpec((2, B, 128), lambda i: (0, i, 0)),
          pl.BlockSpec((2, B, 128), lambda i: (0, i, 0)),
          pl.BlockSpec((H, H), lambda i: (0, 0)),
          pl.BlockSpec((1, H), lambda i: (0, 0)),
          pl.BlockSpec((H, H), lambda i: (0, 0)),
          pl.BlockSpec((1, H), lambda i: (0, 0)),
      ],
      out_specs=pl.BlockSpec((2, B, 128), lambda i: (0, i, 0)),
      out_shape=jax.ShapeDtypeStruct((2, N, 128), jnp.float32),
  )


@functools.lru_cache(maxsize=None)
def _make_mlp2(N, H, B):
  grid = (N // B,)
  return pl.pallas_call(
      _mlp2_body,
      grid=grid,
      in_specs=[
          pl.BlockSpec((2, B, 128), lambda i: (0, i, 0)),
          pl.BlockSpec((2, B, 128), lambda i: (0, i, 0)),
          pl.BlockSpec((H, 128), lambda i: (0, 0)),
          pl.BlockSpec((1, 128), lambda i: (0, 0)),
          pl.BlockSpec((128, 128), lambda i: (0, 0)),
          pl.BlockSpec((1, 128), lambda i: (0, 0)),
      ],
      out_specs=pl.BlockSpec((B, 128), lambda i: (i, 0)),
      out_shape=jax.ShapeDtypeStruct((N, 128), jnp.float32),
  )


def _fold_bn(W1, b1, g, bt, rm, rv):
  sc = g * lax.rsqrt(rv + 1e-5)
  return W1 * sc[None, :], ((b1 - rm) * sc + bt)[None, :]


def kernel(x, edge_index, W1_0, b1_0, g_0, bt_0, rm_0, rv_0, W2_0, b2_0,
           W1_1, b1_1, g_1, bt_1, rm_1, rv_1, W2_1, b2_1,
           W1_2, b1_2, g_2, bt_2, rm_2, rv_2, W2_2, b2_2, eps_0, eps_1):
  N, D = x.shape
  E = edge_index.shape[1]
  H = W1_0.shape[1]
  OUT = W1_2.shape[1]
  assert D == 256 and H == 256 and N % NSUB == 0

  src, dst = edge_index[0], edge_index[1]
  src_all = jnp.concatenate([src, dst])
  dst_all = jnp.concatenate([dst, src])
  E2 = 2 * E
  SB = 16
  NSB = -(-E2 // (NSUB * SB * CH))
  NSB += NSB % 2
  pad = NSUB * NSB * SB * CH - E2
  ACC_ROWS = ((N + 1 + NSUB * 8 - 1) // (NSUB * 8)) * (NSUB * 8)
  zeros = jnp.zeros((ACC_ROWS, 128), jnp.float32)
  ar = jnp.arange(pad, dtype=jnp.int32)
  srcp = jnp.concatenate([src_all, ar % N])
  dstp = jnp.concatenate([dst_all, N + ar % (ACC_ROWS - N)])
  src_r = srcp.reshape(NSUB, NSB, SB, CH)
  src3 = jnp.stack([src_r, src_r + N])
  dst3 = dstp.reshape(NSUB, NSB, SB, CH)
  aggr_fn = _make_aggr(N, NSB, SB, ACC_ROWS)

  W1f0, b1f0 = _fold_bn(W1_0, b1_0, g_0, bt_0, rm_0, rv_0)
  W1f1, b1f1 = _fold_bn(W1_1, b1_1, g_1, bt_1, rm_1, rv_1)
  W1f2, b1f2 = _fold_bn(W1_2, b1_2, g_2, bt_2, rm_2, rv_2)
  W1p = jnp.zeros((H, 128), jnp.float32).at[:, :OUT].set(W1f2)
  b1p = jnp.zeros((1, 128), jnp.float32).at[:, :OUT].set(b1f2)
  W2p = jnp.zeros((128, 128), jnp.float32).at[:OUT, :OUT].set(W2_2)
  b2p = jnp.full((1, 128), -1e30, jnp.float32).at[:, :OUT].set(b2_2[None, :])
  s0 = jnp.reshape(2.0 + eps_0, (1, 1))
  s1 = jnp.reshape(2.0 + eps_1, (1, 1))

  B = 1000
  mlp01 = _make_mlp01(N, H, B)
  mlp2 = _make_mlp2(N, H, B)

  h = jnp.stack([x[:, :128], x[:, 128:]])
  a = aggr_fn(h.reshape(2 * N, 128), src3, dst3, zeros)
  h = mlp01(s0, h, a.reshape(2, N, 128), W1f0, b1f0, W2_0, b2_0[None, :])
  a = aggr_fn(h.reshape(2 * N, 128), src3, dst3, zeros)
  h = mlp01(s1, h, a.reshape(2, N, 128), W1f1, b1f1, W2_1, b2_1[None, :])
  a = aggr_fn(h.reshape(2 * N, 128), src3, dst3, zeros)
  o = mlp2(h, a.reshape(2, N, 128), W1p, b1p, W2p, b2p)
  return o[:, :OUT]


# speedup vs baseline: 16.1590x; 1.1308x over previous
import functools

import jax
import jax.numpy as jnp
from jax import lax
from jax.experimental import pallas as pl
from jax.experimental.pallas import tpu as pltpu
from jax.experimental.pallas import tpu_sc as plsc

NSUB = 16
NCORE = 2
CH = 64


@functools.lru_cache(maxsize=None)
def _make_aggr(N, NSB, SB, ACC_ROWS):
  mesh = plsc.VectorSubcoreMesh(core_axis_name="c", subcore_axis_name="s")
  FR = (N // NSUB) // 8 * 8
  TAIL = N - NSUB * FR
  NZ = 4
  ZR = ACC_ROWS // NZ
  NBUF = 4
  assert NSB % 2 == 0 and NSB >= 4 and SB % NBUF == 0 and SB >= 9
  assert ACC_ROWS % NZ == 0 and ZR % 8 == 0

  @functools.partial(
      pl.kernel,
      out_type=jax.ShapeDtypeStruct((2 * N, 128), jnp.float32),
      mesh=mesh,
      scratch_types=[
          pltpu.VMEM((SB, CH), jnp.int32),
          pltpu.VMEM((SB, CH), jnp.int32),
          pltpu.VMEM((SB, CH), jnp.int32),
          pltpu.VMEM((SB, CH), jnp.int32),
          *[pltpu.VMEM((CH, 128), jnp.float32) for _ in range(NBUF)],
          pltpu.VMEM_SHARED((ACC_ROWS, 128), jnp.float32),
          *[pltpu.SemaphoreType.DMA for _ in range(2 * NBUF)],
          pltpu.SemaphoreType.DMA,
          pltpu.SemaphoreType.DMA,
      ],
  )
  def aggr(h2, src3, dst3, zeros, out, isrcA, isrcB, idstA, idstB, *rest):
    rows = rest[:NBUF]
    acc = rest[NBUF]
    gsem = rest[NBUF + 1:2 * NBUF + 1]
    ssem = rest[2 * NBUF + 1:3 * NBUF + 1]
    isem = rest[3 * NBUF + 1]
    zsem = rest[3 * NBUF + 2]
    cid = lax.axis_index("c")
    sid = lax.axis_index("s")
    isrcs = (isrcA, isrcB)
    idsts = (idstA, idstB)

    @pl.when(sid < NZ)
    def _():
      pltpu.async_copy(zeros.at[pl.ds(sid * ZR, ZR)],
                       acc.at[pl.ds(sid * ZR, ZR)], zsem)
    pltpu.sync_copy(src3.at[cid, sid, 0], isrcA)
    pltpu.sync_copy(dst3.at[sid, 0], idstA)
    pltpu.async_copy(src3.at[cid, sid, 1], isrcB, isem)
    pltpu.async_copy(dst3.at[sid, 1], idstB, isem)
    for j in range(NBUF - 1):
      pltpu.async_copy(h2.at[isrcA.at[j]], rows[j], gsem[j])
    @pl.when(sid < NZ)
    def _():
      pltpu.make_async_copy(zeros.at[pl.ds(0, ZR)],
                            acc.at[pl.ds(0, ZR)], zsem).wait()
    plsc.subcore_barrier()

    def emit_section(kval, s, first):
      cur_isrc, cur_idst = isrcs[s], idsts[s]
      nxt_isrc, nxt_idst = isrcs[1 - s], idsts[1 - s]
      for j in range(SB):
        b = j % NBUF
        if j == 5 and not first:
          knx = jnp.minimum(kval + 1, NSB - 1)
          pltpu.async_copy(src3.at[cid, sid, knx], nxt_isrc, isem)
          pltpu.async_copy(dst3.at[sid, knx], nxt_idst, isem)
        if j == SB - 3:
          pltpu.make_async_copy(src3.at[cid, sid, 0], nxt_isrc, isem).wait()
          pltpu.make_async_copy(dst3.at[sid, 0], nxt_idst, isem).wait()
        if not (first and j == 0):
          pb = (j - 1) % NBUF
          pltpu.make_async_copy(rows[pb], acc.at[cur_idst.at[0]],
                                ssem[pb]).wait()
        pltpu.make_async_copy(h2.at[cur_isrc.at[j]], rows[b], gsem[b]).wait()
        pltpu.async_copy(rows[b], acc.at[cur_idst.at[j]], ssem[b], add=True)
        nb = (j + NBUF - 1) % NBUF
        if j < SB - (NBUF - 1):
          pltpu.async_copy(h2.at[cur_isrc.at[j + NBUF - 1]], rows[nb],
                           gsem[nb])
        else:
          pltpu.async_copy(h2.at[nxt_isrc.at[j + NBUF - 1 - SB]], rows[nb],
                           gsem[nb])

    emit_section(0, 0, True)
    emit_section(1, 1, False)

    def pair(k2, _):
      emit_section(2 * k2, 0, False)
      emit_section(2 * k2 + 1, 1, False)
      return 0

    lax.fori_loop(1, NSB // 2, pair, 0)

    for j in range(NBUF - 1):
      pltpu.make_async_copy(h2.at[isrcA.at[j]], rows[j % NBUF],
                            gsem[j % NBUF]).wait()
    pltpu.make_async_copy(rows[(SB - 1) % NBUF], acc.at[idstA.at[0]],
                          ssem[(SB - 1) % NBUF]).wait()

    plsc.subcore_barrier()
    pltpu.sync_copy(acc.at[pl.ds(sid * FR, FR)],
                    out.at[pl.ds(cid * N + sid * FR, FR)])
    if TAIL:
      @pl.when(sid == NSUB - 1)
      def _():
        pltpu.sync_copy(acc.at[pl.ds(NSUB * FR, TAIL)],
                        out.at[pl.ds(cid * N + NSUB * FR, TAIL)])

  return aggr


def _mlp01_body(s_ref, h_ref, a_ref, w1_ref, b1_ref, w2_ref, b2_ref, o_ref):
  h = jnp.concatenate([h_ref[0], h_ref[1]], axis=-1)
  a = jnp.concatenate([a_ref[0], a_ref[1]], axis=-1)
  z = h * s_ref[0, 0] + a
  t = jnp.dot(z, w1_ref[...], preferred_element_type=jnp.float32)
  t = jnp.maximum(t + b1_ref[...], 0.0)
  o = jnp.dot(t, w2_ref[...], preferred_element_type=jnp.float32)
  o = jnp.maximum(o + b2_ref[...], 0.0)
  o_ref[0] = o[:, :128]
  o_ref[1] = o[:, 128:]


def _mlp2_body(h_ref, a_ref, w1_ref, b1_ref, w2_ref, b2_ref, o_ref):
  h = jnp.concatenate([h_ref[0], h_ref[1]], axis=-1)
  a = jnp.concatenate([a_ref[0], a_ref[1]], axis=-1)
  z = 2.0 * h + a
  t = jnp.dot(z, w1_ref[...], preferred_element_type=jnp.float32)
  t = jnp.maximum(t + b1_ref[...], 0.0)
  o = jnp.dot(t, w2_ref[...], preferred_element_type=jnp.float32)
  o = o + b2_ref[...]
  m = jnp.max(o, axis=-1, keepdims=True)
  lse = jnp.log(jnp.sum(jnp.exp(o - m), axis=-1, keepdims=True)) + m
  o_ref[...] = o - lse


@functools.lru_cache(maxsize=None)
def _make_mlp01(N, H, B):
  grid = (N // B,)
  return pl.pallas_call(
      _mlp01_body,
      grid=grid,
      in_specs=[
          pl.BlockSpec(memory_space=pltpu.SMEM),
          pl.BlockSpec((2, B, 128), lambda i: (0, i, 0)),
          pl.BlockSpec((2, B, 128), lambda i: (0, i, 0)),
          pl.BlockSpec((H, H), lambda i: (0, 0)),
          pl.BlockSpec((1, H), lambda i: (0, 0)),
          pl.BlockSpec((H, H), lambda i: (0, 0)),
          pl.BlockSpec((1, H), lambda i: (0, 0)),
      ],
      out_specs=pl.BlockSpec((2, B, 128), lambda i: (0, i, 0)),
      out_shape=jax.ShapeDtypeStruct((2, N, 128), jnp.float32),
  )


@functools.lru_cache(maxsize=None)
def _make_mlp2(N, H, B):
  grid = (N // B,)
  return pl.pallas_call(
      _mlp2_body,
      grid=grid,
      in_specs=[
          pl.BlockSpec((2, B, 128), lambda i: (0, i, 0)),
          pl.BlockSpec((2, B, 128), lambda i: (0, i, 0)),
          pl.BlockSpec((H, 128), lambda i: (0, 0)),
          pl.BlockSpec((1, 128), lambda i: (0, 0)),
          pl.BlockSpec((128, 128), lambda i: (0, 0)),
          pl.BlockSpec((1, 128), lambda i: (0, 0)),
      ],
      out_specs=pl.BlockSpec((B, 128), lambda i: (i, 0)),
      out_shape=jax.ShapeDtypeStruct((N, 128), jnp.float32),
  )


def _fold_bn(W1, b1, g, bt, rm, rv):
  sc = g * lax.rsqrt(rv + 1e-5)
  return W1 * sc[None, :], ((b1 - rm) * sc + bt)[None, :]


def kernel(x, edge_index, W1_0, b1_0, g_0, bt_0, rm_0, rv_0, W2_0, b2_0,
           W1_1, b1_1, g_1, bt_1, rm_1, rv_1, W2_1, b2_1,
           W1_2, b1_2, g_2, bt_2, rm_2, rv_2, W2_2, b2_2, eps_0, eps_1):
  N, D = x.shape
  E = edge_index.shape[1]
  H = W1_0.shape[1]
  OUT = W1_2.shape[1]
  assert D == 256 and H == 256 and N % NSUB == 0

  src, dst = edge_index[0], edge_index[1]
  src_all = jnp.concatenate([src, dst])
  dst_all = jnp.concatenate([dst, src])
  E2 = 2 * E
  SB = 16
  NSB = -(-E2 // (NSUB * SB * CH))
  NSB += NSB % 2
  pad = NSUB * NSB * SB * CH - E2
  ACC_ROWS = ((N + 1 + NSUB * 8 - 1) // (NSUB * 8)) * (NSUB * 8)
  zeros = jnp.zeros((ACC_ROWS, 128), jnp.float32)
  ar = jnp.arange(pad, dtype=jnp.int32)
  srcp = jnp.concatenate([src_all, ar % N])
  dstp = jnp.concatenate([dst_all, N + ar % (ACC_ROWS - N)])
  src_r = srcp.reshape(NSUB, NSB, SB, CH)
  src3 = jnp.stack([src_r, src_r + N])
  dst3 = dstp.reshape(NSUB, NSB, SB, CH)
  aggr_fn = _make_aggr(N, NSB, SB, ACC_ROWS)

  W1f0, b1f0 = _fold_bn(W1_0, b1_0, g_0, bt_0, rm_0, rv_0)
  W1f1, b1f1 = _fold_bn(W1_1, b1_1, g_1, bt_1, rm_1, rv_1)
  W1f2, b1f2 = _fold_bn(W1_2, b1_2, g_2, bt_2, rm_2, rv_2)
  W1p = jnp.zeros((H, 128), jnp.float32).at[:, :OUT].set(W1f2)
  b1p = jnp.zeros((1, 128), jnp.float32).at[:, :OUT].set(b1f2)
  W2p = jnp.zeros((128, 128), jnp.float32).at[:OUT, :OUT].set(W2_2)
  b2p = jnp.full((1, 128), -1e30, jnp.float32).at[:, :OUT].set(b2_2[None, :])
  s0 = jnp.reshape(2.0 + eps_0, (1, 1))
  s1 = jnp.reshape(2.0 + eps_1, (1, 1))

  B = 1000
  mlp01 = _make_mlp01(N, H, B)
  mlp2 = _make_mlp2(N, H, B)

  h = jnp.stack([x[:, :128], x[:, 128:]])
  a = aggr_fn(h.reshape(2 * N, 128), src3, dst3, zeros)
  h = mlp01(s0, h, a.reshape(2, N, 128), W1f0, b1f0, W2_0, b2_0[None, :])
  a = aggr_fn(h.reshape(2 * N, 128), src3, dst3, zeros)
  h = mlp01(s1, h, a.reshape(2, N, 128), W1f1, b1f1, W2_1, b2_1[None, :])
  a = aggr_fn(h.reshape(2 * N, 128), src3, dst3, zeros)
  o = mlp2(h, a.reshape(2, N, 128), W1p, b1p, W2p, b2p)
  return o[:, :OUT]


# speedup vs baseline: 16.3213x; 1.0100x over previous
import functools

import jax
import jax.numpy as jnp
from jax import lax
from jax.experimental import pallas as pl
from jax.experimental.pallas import tpu as pltpu
from jax.experimental.pallas import tpu_sc as plsc

NSUB = 16
NCORE = 2
CH = 80


@functools.lru_cache(maxsize=None)
def _make_aggr(N, NSB, SB, ACC_ROWS):
  mesh = plsc.VectorSubcoreMesh(core_axis_name="c", subcore_axis_name="s")
  FR = (N // NSUB) // 8 * 8
  TAIL = N - NSUB * FR
  NZ = 4
  ZR = ACC_ROWS // NZ
  NBUF = 4
  assert NSB % 2 == 0 and NSB >= 4 and SB % NBUF == 0 and SB >= 9
  assert ACC_ROWS % NZ == 0 and ZR % 8 == 0

  @functools.partial(
      pl.kernel,
      out_type=jax.ShapeDtypeStruct((2 * N, 128), jnp.float32),
      mesh=mesh,
      scratch_types=[
          pltpu.VMEM((SB, CH), jnp.int32),
          pltpu.VMEM((SB, CH), jnp.int32),
          pltpu.VMEM((SB, CH), jnp.int32),
          pltpu.VMEM((SB, CH), jnp.int32),
          *[pltpu.VMEM((CH, 128), jnp.float32) for _ in range(NBUF)],
          pltpu.VMEM_SHARED((ACC_ROWS, 128), jnp.float32),
          *[pltpu.SemaphoreType.DMA for _ in range(2 * NBUF)],
          pltpu.SemaphoreType.DMA,
          pltpu.SemaphoreType.DMA,
      ],
  )
  def aggr(h2, src3, dst3, zeros, out, isrcA, isrcB, idstA, idstB, *rest):
    rows = rest[:NBUF]
    acc = rest[NBUF]
    gsem = rest[NBUF + 1:2 * NBUF + 1]
    ssem = rest[2 * NBUF + 1:3 * NBUF + 1]
    isem = rest[3 * NBUF + 1]
    zsem = rest[3 * NBUF + 2]
    cid = lax.axis_index("c")
    sid = lax.axis_index("s")
    isrcs = (isrcA, isrcB)
    idsts = (idstA, idstB)

    @pl.when(sid < NZ)
    def _():
      pltpu.async_copy(zeros.at[pl.ds(sid * ZR, ZR)],
                       acc.at[pl.ds(sid * ZR, ZR)], zsem)
    pltpu.sync_copy(src3.at[cid, sid, 0], isrcA)
    pltpu.sync_copy(dst3.at[sid, 0], idstA)
    pltpu.async_copy(src3.at[cid, sid, 1], isrcB, isem)
    pltpu.async_copy(dst3.at[sid, 1], idstB, isem)
    for j in range(NBUF - 1):
      pltpu.async_copy(h2.at[isrcA.at[j]], rows[j], gsem[j])
    @pl.when(sid < NZ)
    def _():
      pltpu.make_async_copy(zeros.at[pl.ds(0, ZR)],
                            acc.at[pl.ds(0, ZR)], zsem).wait()
    plsc.subcore_barrier()

    def emit_section(kval, s, first):
      cur_isrc, cur_idst = isrcs[s], idsts[s]
      nxt_isrc, nxt_idst = isrcs[1 - s], idsts[1 - s]
      for j in range(SB):
        b = j % NBUF
        if j == 5 and not first:
          knx = jnp.minimum(kval + 1, NSB - 1)
          pltpu.async_copy(src3.at[cid, sid, knx], nxt_isrc, isem)
          pltpu.async_copy(dst3.at[sid, knx], nxt_idst, isem)
        if j == SB - 3:
          pltpu.make_async_copy(src3.at[cid, sid, 0], nxt_isrc, isem).wait()
          pltpu.make_async_copy(dst3.at[sid, 0], nxt_idst, isem).wait()
        if not (first and j == 0):
          pb = (j - 1) % NBUF
          pltpu.make_async_copy(rows[pb], acc.at[cur_idst.at[0]],
                                ssem[pb]).wait()
        pltpu.make_async_copy(h2.at[cur_isrc.at[j]], rows[b], gsem[b]).wait()
        pltpu.async_copy(rows[b], acc.at[cur_idst.at[j]], ssem[b], add=True)
        nb = (j + NBUF - 1) % NBUF
        if j < SB - (NBUF - 1):
          pltpu.async_copy(h2.at[cur_isrc.at[j + NBUF - 1]], rows[nb],
                           gsem[nb])
        else:
          pltpu.async_copy(h2.at[nxt_isrc.at[j + NBUF - 1 - SB]], rows[nb],
                           gsem[nb])

    emit_section(0, 0, True)
    emit_section(1, 1, False)

    def pair(k2, _):
      emit_section(2 * k2, 0, False)
      emit_section(2 * k2 + 1, 1, False)
      return 0

    lax.fori_loop(1, NSB // 2, pair, 0)

    for j in range(NBUF - 1):
      pltpu.make_async_copy(h2.at[isrcA.at[j]], rows[j % NBUF],
                            gsem[j % NBUF]).wait()
    pltpu.make_async_copy(rows[(SB - 1) % NBUF], acc.at[idstA.at[0]],
                          ssem[(SB - 1) % NBUF]).wait()

    plsc.subcore_barrier()
    pltpu.sync_copy(acc.at[pl.ds(sid * FR, FR)],
                    out.at[pl.ds(cid * N + sid * FR, FR)])
    if TAIL:
      @pl.when(sid == NSUB - 1)
      def _():
        pltpu.sync_copy(acc.at[pl.ds(NSUB * FR, TAIL)],
                        out.at[pl.ds(cid * N + NSUB * FR, TAIL)])

  return aggr


def _mlp01_body(s_ref, h_ref, a_ref, w1_ref, b1_ref, w2_ref, b2_ref, o_ref):
  h = jnp.concatenate([h_ref[0], h_ref[1]], axis=-1)
  a = jnp.concatenate([a_ref[0], a_ref[1]], axis=-1)
  z = h * s_ref[0, 0] + a
  t = jnp.dot(z, w1_ref[...], preferred_element_type=jnp.float32)
  t = jnp.maximum(t + b1_ref[...], 0.0)
  o = jnp.dot(t, w2_ref[...], preferred_element_type=jnp.float32)
  o = jnp.maximum(o + b2_ref[...], 0.0)
  o_ref[0] = o[:, :128]
  o_ref[1] = o[:, 128:]


def _mlp2_body(h_ref, a_ref, w1_ref, b1_ref, w2_ref, b2_ref, o_ref):
  h = jnp.concatenate([h_ref[0], h_ref[1]], axis=-1)
  a = jnp.concatenate([a_ref[0], a_ref[1]], axis=-1)
  z = 2.0 * h + a
  t = jnp.dot(z, w1_ref[...], preferred_element_type=jnp.float32)
  t = jnp.maximum(t + b1_ref[...], 0.0)
  o = jnp.dot(t, w2_ref[...], preferred_element_type=jnp.float32)
  o = o + b2_ref[...]
  m = jnp.max(o, axis=-1, keepdims=True)
  lse = jnp.log(jnp.sum(jnp.exp(o - m), axis=-1, keepdims=True)) + m
  o_ref[...] = o - lse


@functools.lru_cache(maxsize=None)
def _make_mlp01(N, H, B):
  grid = (N // B,)
  return pl.pallas_call(
      _mlp01_body,
      grid=grid,
      in_specs=[
          pl.BlockSpec(memory_space=pltpu.SMEM),
          pl.BlockSpec((2, B, 128), lambda i: (0, i, 0)),
          pl.BlockSpec((2, B, 128), lambda i: (0, i, 0)),
          pl.BlockSpec((H, H), lambda i: (0, 0)),
          pl.BlockSpec((1, H), lambda i: (0, 0)),
          pl.BlockSpec((H, H), lambda i: (0, 0)),
          pl.BlockSpec((1, H), lambda i: (0, 0)),
      ],
      out_specs=pl.BlockSpec((2, B, 128), lambda i: (0, i, 0)),
      out_shape=jax.ShapeDtypeStruct((2, N, 128), jnp.float32),
  )


@functools.lru_cache(maxsize=None)
def _make_mlp2(N, H, B):
  grid = (N // B,)
  return pl.pallas_call(
      _mlp2_body,
      grid=grid,
      in_specs=[
          pl.BlockSpec((2, B, 128), lambda i: (0, i, 0)),
          pl.BlockSpec((2, B, 128), lambda i: (0, i, 0)),
          pl.BlockSpec((H, 128), lambda i: (0, 0)),
          pl.BlockSpec((1, 128), lambda i: (0, 0)),
          pl.BlockSpec((128, 128), lambda i: (0, 0)),
          pl.BlockSpec((1, 128), lambda i: (0, 0)),
      ],
      out_specs=pl.BlockSpec((B, 128), lambda i: (i, 0)),
      out_shape=jax.ShapeDtypeStruct((N, 128), jnp.float32),
  )


def _fold_bn(W1, b1, g, bt, rm, rv):
  sc = g * lax.rsqrt(rv + 1e-5)
  return W1 * sc[None, :], ((b1 - rm) * sc + bt)[None, :]


def kernel(x, edge_index, W1_0, b1_0, g_0, bt_0, rm_0, rv_0, W2_0, b2_0,
           W1_1, b1_1, g_1, bt_1, rm_1, rv_1, W2_1, b2_1,
           W1_2, b1_2, g_2, bt_2, rm_2, rv_2, W2_2, b2_2, eps_0, eps_1):
  N, D = x.shape
  E = edge_index.shape[1]
  H = W1_0.shape[1]
  OUT = W1_2.shape[1]
  assert D == 256 and H == 256 and N % NSUB == 0

  src, dst = edge_index[0], edge_index[1]
  src_all = jnp.concatenate([src, dst])
  dst_all = jnp.concatenate([dst, src])
  E2 = 2 * E
  SB = 16
  NSB = -(-E2 // (NSUB * SB * CH))
  NSB += NSB % 2
  pad = NSUB * NSB * SB * CH - E2
  ACC_ROWS = ((N + 1 + NSUB * 8 - 1) // (NSUB * 8)) * (NSUB * 8)
  zeros = jnp.zeros((ACC_ROWS, 128), jnp.float32)
  ar = jnp.arange(pad, dtype=jnp.int32)
  srcp = jnp.concatenate([src_all, ar % N])
  dstp = jnp.concatenate([dst_all, N + ar % (ACC_ROWS - N)])
  src_r = srcp.reshape(NSUB, NSB, SB, CH)
  src3 = jnp.stack([src_r, src_r + N])
  dst3 = dstp.reshape(NSUB, NSB, SB, CH)
  aggr_fn = _make_aggr(N, NSB, SB, ACC_ROWS)

  W1f0, b1f0 = _fold_bn(W1_0, b1_0, g_0, bt_0, rm_0, rv_0)
  W1f1, b1f1 = _fold_bn(W1_1, b1_1, g_1, bt_1, rm_1, rv_1)
  W1f2, b1f2 = _fold_bn(W1_2, b1_2, g_2, bt_2, rm_2, rv_2)
  W1p = jnp.zeros((H, 128), jnp.float32).at[:, :OUT].set(W1f2)
  b1p = jnp.zeros((1, 128), jnp.float32).at[:, :OUT].set(b1f2)
  W2p = jnp.zeros((128, 128), jnp.float32).at[:OUT, :OUT].set(W2_2)
  b2p = jnp.full((1, 128), -1e30, jnp.float32).at[:, :OUT].set(b2_2[None, :])
  s0 = jnp.reshape(2.0 + eps_0, (1, 1))
  s1 = jnp.reshape(2.0 + eps_1, (1, 1))

  B = 1000
  mlp01 = _make_mlp01(N, H, B)
  mlp2 = _make_mlp2(N, H, B)

  h = jnp.stack([x[:, :128], x[:, 128:]])
  a = aggr_fn(h.reshape(2 * N, 128), src3, dst3, zeros)
  h = mlp01(s0, h, a.reshape(2, N, 128), W1f0, b1f0, W2_0, b2_0[None, :])
  a = aggr_fn(h.reshape(2 * N, 128), src3, dst3, zeros)
  h = mlp01(s1, h, a.reshape(2, N, 128), W1f1, b1f1, W2_1, b2_1[None, :])
  a = aggr_fn(h.reshape(2 * N, 128), src3, dst3, zeros)
  o = mlp2(h, a.reshape(2, N, 128), W1p, b1p, W2p, b2p)
  return o[:, :OUT]


# speedup vs baseline: 16.4891x; 1.0103x over previous
import functools

import jax
import jax.numpy as jnp
from jax import lax
from jax.experimental import pallas as pl
from jax.experimental.pallas import tpu as pltpu
from jax.experimental.pallas import tpu_sc as plsc

NSUB = 16
NCORE = 2
CH = 80


@functools.lru_cache(maxsize=None)
def _make_aggr(N, NSB, SB, ACC_ROWS):
  mesh = plsc.VectorSubcoreMesh(core_axis_name="c", subcore_axis_name="s")
  FR = (N // NSUB) // 8 * 8
  TAIL = N - NSUB * FR
  NZ = 4
  ZR = ACC_ROWS // NZ
  NBUF = 4
  assert NSB % 2 == 0 and NSB >= 4 and SB % NBUF == 0 and SB >= 9
  assert ACC_ROWS % NZ == 0 and ZR % 8 == 0

  @functools.partial(
      pl.kernel,
      out_type=jax.ShapeDtypeStruct((2 * N, 128), jnp.float32),
      mesh=mesh,
      scratch_types=[
          pltpu.VMEM((SB, CH), jnp.int32),
          pltpu.VMEM((SB, CH), jnp.int32),
          pltpu.VMEM((SB, CH), jnp.int32),
          pltpu.VMEM((SB, CH), jnp.int32),
          *[pltpu.VMEM((CH, 128), jnp.float32) for _ in range(NBUF)],
          pltpu.VMEM_SHARED((ACC_ROWS, 128), jnp.float32),
          *[pltpu.SemaphoreType.DMA for _ in range(2 * NBUF)],
          pltpu.SemaphoreType.DMA,
          pltpu.SemaphoreType.DMA,
      ],
  )
  def aggr(h2, src3, dst3, zeros, out, isrcA, isrcB, idstA, idstB, *rest):
    rows = rest[:NBUF]
    acc = rest[NBUF]
    gsem = rest[NBUF + 1:2 * NBUF + 1]
    ssem = rest[2 * NBUF + 1:3 * NBUF + 1]
    isem = rest[3 * NBUF + 1]
    zsem = rest[3 * NBUF + 2]
    cid = lax.axis_index("c")
    sid = lax.axis_index("s")
    isrcs = (isrcA, isrcB)
    idsts = (idstA, idstB)

    @pl.when(sid < NZ)
    def _():
      pltpu.async_copy(zeros.at[pl.ds(sid * ZR, ZR)],
                       acc.at[pl.ds(sid * ZR, ZR)], zsem)
    pltpu.sync_copy(src3.at[cid, sid, 0], isrcA)
    pltpu.sync_copy(dst3.at[sid, 0], idstA)
    pltpu.async_copy(src3.at[cid, sid, 1], isrcB, isem)
    pltpu.async_copy(dst3.at[sid, 1], idstB, isem)
    for j in range(NBUF - 1):
      pltpu.async_copy(h2.at[isrcA.at[j]], rows[j], gsem[j])
    @pl.when(sid < NZ)
    def _():
      pltpu.make_async_copy(zeros.at[pl.ds(0, ZR)],
                            acc.at[pl.ds(0, ZR)], zsem).wait()
    plsc.subcore_barrier()

    def emit_section(kval, s, first):
      cur_isrc, cur_idst = isrcs[s], idsts[s]
      nxt_isrc, nxt_idst = isrcs[1 - s], idsts[1 - s]
      for j in range(SB):
        b = j % NBUF
        if j == 5 and not first:
          knx = jnp.minimum(kval + 1, NSB - 1)
          pltpu.async_copy(src3.at[cid, sid, knx], nxt_isrc, isem)
          pltpu.async_copy(dst3.at[sid, knx], nxt_idst, isem)
        if j == SB - 3:
          pltpu.make_async_copy(src3.at[cid, sid, 0], nxt_isrc, isem).wait()
          pltpu.make_async_copy(dst3.at[sid, 0], nxt_idst, isem).wait()
        if not (first and j == 0):
          pb = (j - 1) % NBUF
          pltpu.make_async_copy(rows[pb], acc.at[cur_idst.at[0]],
                                ssem[pb]).wait()
        pltpu.make_async_copy(h2.at[cur_isrc.at[j]], rows[b], gsem[b]).wait()
        pltpu.async_copy(rows[b], acc.at[cur_idst.at[j]], ssem[b], add=True)
        nb = (j + NBUF - 1) % NBUF
        if j < SB - (NBUF - 1):
          pltpu.async_copy(h2.at[cur_isrc.at[j + NBUF - 1]], rows[nb],
                           gsem[nb])
        else:
          pltpu.async_copy(h2.at[nxt_isrc.at[j + NBUF - 1 - SB]], rows[nb],
                           gsem[nb])

    emit_section(0, 0, True)
    emit_section(1, 1, False)

    def pair(k2, _):
      emit_section(2 * k2, 0, False)
      emit_section(2 * k2 + 1, 1, False)
      return 0

    lax.fori_loop(1, NSB // 2, pair, 0)

    for j in range(NBUF - 1):
      pltpu.make_async_copy(h2.at[isrcA.at[j]], rows[j % NBUF],
                            gsem[j % NBUF]).wait()
    pltpu.make_async_copy(rows[(SB - 1) % NBUF], acc.at[idstA.at[0]],
                          ssem[(SB - 1) % NBUF]).wait()

    plsc.subcore_barrier()
    pltpu.sync_copy(acc.at[pl.ds(sid * FR, FR)],
                    out.at[pl.ds(cid * N + sid * FR, FR)])
    if TAIL:
      @pl.when(sid == NSUB - 1)
      def _():
        pltpu.sync_copy(acc.at[pl.ds(NSUB * FR, TAIL)],
                        out.at[pl.ds(cid * N + NSUB * FR, TAIL)])

  return aggr


def _mlp01_body(s_ref, h_ref, a_ref, w1_ref, b1_ref, w2_ref, b2_ref, o_ref):
  h = jnp.concatenate([h_ref[0], h_ref[1]], axis=-1)
  a = jnp.concatenate([a_ref[0], a_ref[1]], axis=-1)
  z = h * s_ref[0, 0] + a
  t = jnp.dot(z, w1_ref[...], preferred_element_type=jnp.float32)
  t = jnp.maximum(t + b1_ref[...], 0.0)
  o = jnp.dot(t, w2_ref[...], preferred_element_type=jnp.float32)
  o = jnp.maximum(o + b2_ref[...], 0.0)
  o_ref[0] = o[:, :128]
  o_ref[1] = o[:, 128:]


def _mlp2_body(h_ref, a_ref, w1_ref, b1_ref, w2_ref, b2_ref, o_ref):
  h = jnp.concatenate([h_ref[0], h_ref[1]], axis=-1)
  a = jnp.concatenate([a_ref[0], a_ref[1]], axis=-1)
  z = 2.0 * h + a
  t = jnp.dot(z, w1_ref[...], preferred_element_type=jnp.float32)
  t = jnp.maximum(t + b1_ref[...], 0.0)
  o = jnp.dot(t, w2_ref[...], preferred_element_type=jnp.float32)
  o = o + b2_ref[...]
  m = jnp.max(o, axis=-1, keepdims=True)
  lse = jnp.log(jnp.sum(jnp.exp(o - m), axis=-1, keepdims=True)) + m
  o_ref[...] = o - lse


@functools.lru_cache(maxsize=None)
def _make_mlp01(N, H, B):
  grid = (N // B,)
  return pl.pallas_call(
      _mlp01_body,
      grid=grid,
      in_specs=[
          pl.BlockSpec(memory_space=pltpu.SMEM),
          pl.BlockSpec((2, B, 128), lambda i: (0, i, 0)),
          pl.BlockSpec((2, B, 128), lambda i: (0, i, 0)),
          pl.BlockSpec((H, H), lambda i: (0, 0)),
          pl.BlockSpec((1, H), lambda i: (0, 0)),
          pl.BlockSpec((H, H), lambda i: (0, 0)),
          pl.BlockSpec((1, H), lambda i: (0, 0)),
      ],
      out_specs=pl.BlockSpec((2, B, 128), lambda i: (0, i, 0)),
      out_shape=jax.ShapeDtypeStruct((2, N, 128), jnp.float32),
  )


@functools.lru_cache(maxsize=None)
def _make_mlp2(N, H, B):
  grid = (N // B,)
  return pl.pallas_call(
      _mlp2_body,
      grid=grid,
      in_specs=[
          pl.BlockSpec((2, B, 128), lambda i: (0, i, 0)),
          pl.BlockSpec((2, B, 128), lambda i: (0, i, 0)),
          pl.BlockSpec((H, 128), lambda i: (0, 0)),
          pl.BlockSpec((1, 128), lambda i: (0, 0)),
          pl.BlockSpec((128, 128), lambda i: (0, 0)),
          pl.BlockSpec((1, 128), lambda i: (0, 0)),
      ],
      out_specs=pl.BlockSpec((B, 128), lambda i: (i, 0)),
      out_shape=jax.ShapeDtypeStruct((N, 128), jnp.float32),
  )


def _fold_bn(W1, b1, g, bt, rm, rv):
  sc = g * lax.rsqrt(rv + 1e-5)
  return W1 * sc[None, :], ((b1 - rm) * sc + bt)[None, :]


def kernel(x, edge_index, W1_0, b1_0, g_0, bt_0, rm_0, rv_0, W2_0, b2_0,
           W1_1, b1_1, g_1, bt_1, rm_1, rv_1, W2_1, b2_1,
           W1_2, b1_2, g_2, bt_2, rm_2, rv_2, W2_2, b2_2, eps_0, eps_1):
  N, D = x.shape
  E = edge_index.shape[1]
  H = W1_0.shape[1]
  OUT = W1_2.shape[1]
  assert D == 256 and H == 256 and N % NSUB == 0

  src, dst = edge_index[0], edge_index[1]
  src_all = jnp.concatenate([src, dst])
  dst_all = jnp.concatenate([dst, src])
  E2 = 2 * E
  SB = 16
  NSB = -(-E2 // (NSUB * SB * CH))
  NSB += NSB % 2
  pad = NSUB * NSB * SB * CH - E2
  ACC_ROWS = ((N + 1 + NSUB * 8 - 1) // (NSUB * 8)) * (NSUB * 8)
  zeros = jnp.zeros((ACC_ROWS, 128), jnp.float32)
  ar = jnp.arange(pad, dtype=jnp.int32)
  srcp = jnp.concatenate([src_all, ar % N])
  dstp = jnp.concatenate([dst_all, N + ar % (ACC_ROWS - N)])
  src_r = srcp.reshape(NSUB, NSB, SB, CH)
  src3 = jnp.stack([src_r, src_r + N])
  dst3 = dstp.reshape(NSUB, NSB, SB, CH)
  aggr_fn = _make_aggr(N, NSB, SB, ACC_ROWS)

  W1f0, b1f0 = _fold_bn(W1_0, b1_0, g_0, bt_0, rm_0, rv_0)
  W1f1, b1f1 = _fold_bn(W1_1, b1_1, g_1, bt_1, rm_1, rv_1)
  W1f2, b1f2 = _fold_bn(W1_2, b1_2, g_2, bt_2, rm_2, rv_2)
  W1p = jnp.zeros((H, 128), jnp.float32).at[:, :OUT].set(W1f2)
  b1p = jnp.zeros((1, 128), jnp.float32).at[:, :OUT].set(b1f2)
  W2p = jnp.zeros((128, 128), jnp.float32).at[:OUT, :OUT].set(W2_2)
  b2p = jnp.full((1, 128), -1e30, jnp.float32).at[:, :OUT].set(b2_2[None, :])
  s0 = jnp.reshape(2.0 + eps_0, (1, 1))
  s1 = jnp.reshape(2.0 + eps_1, (1, 1))

  B = 2000
  mlp01 = _make_mlp01(N, H, B)
  mlp2 = _make_mlp2(N, H, B)

  h = jnp.stack([x[:, :128], x[:, 128:]])
  a = aggr_fn(h.reshape(2 * N, 128), src3, dst3, zeros)
  h = mlp01(s0, h, a.reshape(2, N, 128), W1f0, b1f0, W2_0, b2_0[None, :])
  a = aggr_fn(h.reshape(2 * N, 128), src3, dst3, zeros)
  h = mlp01(s1, h, a.reshape(2, N, 128), W1f1, b1f1, W2_1, b2_1[None, :])
  a = aggr_fn(h.reshape(2 * N, 128), src3, dst3, zeros)
  o = mlp2(h, a.reshape(2, N, 128), W1p, b1p, W2p, b2p)
  return o[:, :OUT]
